# Initial kernel scaffold; baseline (speedup 1.0000x reference)
import jax
import jax.numpy as jnp
from jax.experimental import pallas as pl


def kernel(diagrams, masks, grid, grid_bounds):
    raise NotImplementedError("write your pallas kernel here")


# speedup vs baseline: 9.0426x; 9.0426x over previous
import functools

import jax
import jax.numpy as jnp
from jax import lax
from jax.experimental import pallas as pl
from jax.experimental.pallas import tpu as pltpu
from jax.experimental.pallas import tpu_sc as plsc

_N_DIAG = 4096
_F_PTS = 200
_GRID_N = 128
_P = _N_DIAG * _F_PTS
_NW = 32
_PER_W = _P // _NW
_LANES = 16
_ITERS = _PER_W // _LANES

_mesh = plsc.VectorSubcoreMesh(core_axis_name="c", subcore_axis_name="s")


@functools.partial(
    pl.kernel,
    mesh=_mesh,
    out_type=jax.ShapeDtypeStruct((_P,), jnp.float32),
    compiler_params=pltpu.CompilerParams(needs_layout_passes=False),
    scratch_types=[
        pltpu.VMEM((2 * _PER_W,), jnp.float32),
        pltpu.VMEM((_GRID_N, _GRID_N), jnp.float32),
        pltpu.VMEM((_PER_W,), jnp.float32),
        pltpu.VMEM((64,), jnp.float32),
    ],
)
def _sc_lookup(coords_hbm, grid_hbm, params_hbm, out_hbm,
               coords_v, grid_v, out_v, params_v):
    wid = lax.axis_index("s") * 2 + lax.axis_index("c")
    base = wid * (2 * _PER_W)
    pltpu.sync_copy(coords_hbm.at[pl.ds(base, 2 * _PER_W)], coords_v)
    pltpu.sync_copy(grid_hbm, grid_v)
    pltpu.sync_copy(params_hbm, params_v)

    m0 = params_v[pl.ds(0, 16)]
    sx = params_v[pl.ds(16, 16)]
    m1 = params_v[pl.ds(32, 16)]
    sy = params_v[pl.ds(48, 16)]
    even = lax.iota(jnp.int32, 16) * 2

    def body(i, carry):
        xi = even + i * 32
        xs = plsc.load_gather(coords_v, [xi])
        ys = plsc.load_gather(coords_v, [xi + 1])
        ix = jnp.clip(((xs - m0) * sx).astype(jnp.int32), 0, _GRID_N - 1)
        iy = jnp.clip(((ys - m1) * sy).astype(jnp.int32), 0, _GRID_N - 1)
        out_v[pl.ds(i * 16, 16)] = plsc.load_gather(grid_v, [ix, iy])
        return carry

    lax.fori_loop(0, _ITERS, body, 0)
    pltpu.sync_copy(out_v, out_hbm.at[pl.ds(wid * _PER_W, _PER_W)])


def kernel(diagrams, masks, grid, grid_bounds):
    del masks
    coords = diagrams.reshape(-1)
    m0, big0 = grid_bounds[0, 0], grid_bounds[0, 1]
    m1, big1 = grid_bounds[1, 0], grid_bounds[1, 1]
    sx = _GRID_N / (big0 - m0)
    sy = _GRID_N / (big1 - m1)
    params = jnp.concatenate([
        jnp.full((16,), m0, jnp.float32), jnp.full((16,), sx, jnp.float32),
        jnp.full((16,), m1, jnp.float32), jnp.full((16,), sy, jnp.float32)])
    out = _sc_lookup(coords, grid, params)
    return out.reshape(_N_DIAG, _F_PTS, 1)


# speedup vs baseline: 155.2440x; 17.1682x over previous
import functools

import jax
import jax.numpy as jnp
from jax import lax
from jax.experimental import pallas as pl
from jax.experimental.pallas import tpu as pltpu
from jax.experimental.pallas import tpu_sc as plsc

_N_DIAG = 4096
_F_PTS = 200
_GRID_N = 128
_NW = 32
_NCHUNK = _N_DIAG // _NW
_LANES = 16

_mesh = plsc.VectorSubcoreMesh(core_axis_name="c", subcore_axis_name="s")


@functools.partial(
    pl.kernel,
    mesh=_mesh,
    out_type=jax.ShapeDtypeStruct((_F_PTS, _N_DIAG), jnp.float32),
    compiler_params=pltpu.CompilerParams(needs_layout_passes=False),
    scratch_types=[
        pltpu.VMEM((_F_PTS, 2, _NCHUNK), jnp.float32),
        pltpu.VMEM((_GRID_N, _GRID_N), jnp.float32),
        pltpu.VMEM((_F_PTS, _NCHUNK), jnp.float32),
        pltpu.VMEM((64,), jnp.float32),
    ],
)
def _sc_lookup(coords_hbm, grid_hbm, params_hbm, out_hbm,
               coords_v, grid_v, out_v, params_v):
    wid = lax.axis_index("s") * 2 + lax.axis_index("c")
    n0 = wid * _NCHUNK
    pltpu.sync_copy(coords_hbm.at[:, :, pl.ds(n0, _NCHUNK)], coords_v)
    pltpu.sync_copy(grid_hbm, grid_v)
    pltpu.sync_copy(params_hbm, params_v)

    m0 = params_v[pl.ds(0, 16)]
    sx = params_v[pl.ds(16, 16)]
    m1 = params_v[pl.ds(32, 16)]
    sy = params_v[pl.ds(48, 16)]

    def body(f, carry):
        for g in range(_NCHUNK // _LANES):
            xs = coords_v[f, 0, pl.ds(g * _LANES, _LANES)]
            ys = coords_v[f, 1, pl.ds(g * _LANES, _LANES)]
            ix = jnp.clip(((xs - m0) * sx).astype(jnp.int32), 0, _GRID_N - 1)
            iy = jnp.clip(((ys - m1) * sy).astype(jnp.int32), 0, _GRID_N - 1)
            out_v[f, pl.ds(g * _LANES, _LANES)] = plsc.load_gather(
                grid_v, [ix, iy])
        return carry

    lax.fori_loop(0, _F_PTS, body, 0)
    pltpu.sync_copy(out_v, out_hbm.at[:, pl.ds(n0, _NCHUNK)])


def kernel(diagrams, masks, grid, grid_bounds):
    del masks
    coords = jnp.transpose(diagrams, (1, 2, 0))
    m0, big0 = grid_bounds[0, 0], grid_bounds[0, 1]
    m1, big1 = grid_bounds[1, 0], grid_bounds[1, 1]
    sx = _GRID_N / (big0 - m0)
    sy = _GRID_N / (big1 - m1)
    params = jnp.concatenate([
        jnp.full((16,), m0, jnp.float32), jnp.full((16,), sx, jnp.float32),
        jnp.full((16,), m1, jnp.float32), jnp.full((16,), sy, jnp.float32)])
    out = _sc_lookup(coords, grid, params)
    return jnp.transpose(out, (1, 0)).reshape(_N_DIAG, _F_PTS, 1)


# speedup vs baseline: 237.4990x; 1.5298x over previous
import functools

import jax
import jax.numpy as jnp
from jax import lax
from jax.experimental import pallas as pl
from jax.experimental.pallas import tpu as pltpu
from jax.experimental.pallas import tpu_sc as plsc

_N_DIAG = 4096
_F_PTS = 200
_GRID_N = 128
_NW = 32
_NCHUNK = _N_DIAG // _NW
_LANES = 16

_mesh = plsc.VectorSubcoreMesh(core_axis_name="c", subcore_axis_name="s")


@functools.partial(
    pl.kernel,
    mesh=_mesh,
    out_type=jax.ShapeDtypeStruct((_F_PTS, _N_DIAG), jnp.float32),
    compiler_params=pltpu.CompilerParams(needs_layout_passes=False),
    scratch_types=[
        pltpu.VMEM((_F_PTS, 2, _NCHUNK), jnp.float32),
        pltpu.VMEM((_GRID_N, _GRID_N), jnp.float32),
        pltpu.VMEM((_F_PTS, _NCHUNK), jnp.float32),
        pltpu.VMEM((64,), jnp.float32),
    ],
)
def _sc_lookup(coords_hbm, grid_hbm, params_hbm, out_hbm,
               coords_v, grid_v, out_v, params_v):
    wid = lax.axis_index("s") * 2 + lax.axis_index("c")
    n0 = wid * _NCHUNK
    pltpu.sync_copy(coords_hbm.at[:, :, pl.ds(n0, _NCHUNK)], coords_v)
    pltpu.sync_copy(grid_hbm, grid_v)
    pltpu.sync_copy(params_hbm, params_v)

    m0 = params_v[pl.ds(0, 16)]
    sx = params_v[pl.ds(16, 16)]
    m1 = params_v[pl.ds(32, 16)]
    sy = params_v[pl.ds(48, 16)]
    lim = jnp.full((_LANES,), float(_GRID_N - 1), jnp.float32)
    zero = jnp.zeros((_LANES,), jnp.float32)

    @plsc.parallel_loop(0, _F_PTS, unroll=2)
    def _loop(f):
        idx = []
        for g in range(_NCHUNK // _LANES):
            xs = coords_v[f, 0, pl.ds(g * _LANES, _LANES)]
            ys = coords_v[f, 1, pl.ds(g * _LANES, _LANES)]
            fx = jnp.minimum(jnp.maximum((xs - m0) * sx, zero), lim)
            fy = jnp.minimum(jnp.maximum((ys - m1) * sy, zero), lim)
            idx.append((fx.astype(jnp.int32), fy.astype(jnp.int32)))
        ws = [plsc.load_gather(grid_v, [ix, iy]) for ix, iy in idx]
        for g, w in enumerate(ws):
            out_v[f, pl.ds(g * _LANES, _LANES)] = w
    pltpu.sync_copy(out_v, out_hbm.at[:, pl.ds(n0, _NCHUNK)])


def kernel(diagrams, masks, grid, grid_bounds):
    del masks
    coords = jnp.transpose(diagrams, (1, 2, 0))
    m0, big0 = grid_bounds[0, 0], grid_bounds[0, 1]
    m1, big1 = grid_bounds[1, 0], grid_bounds[1, 1]
    sx = _GRID_N / (big0 - m0)
    sy = _GRID_N / (big1 - m1)
    params = jnp.concatenate([
        jnp.full((16,), m0, jnp.float32), jnp.full((16,), sx, jnp.float32),
        jnp.full((16,), m1, jnp.float32), jnp.full((16,), sy, jnp.float32)])
    out = _sc_lookup(coords, grid, params)
    return jnp.transpose(out, (1, 0)).reshape(_N_DIAG, _F_PTS, 1)


# speedup vs baseline: 256.6179x; 1.0805x over previous
import functools

import jax
import jax.numpy as jnp
from jax import lax
from jax.experimental import pallas as pl
from jax.experimental.pallas import tpu as pltpu
from jax.experimental.pallas import tpu_sc as plsc

_N_DIAG = 4096
_F_PTS = 200
_GRID_N = 128
_NW = 32
_NCHUNK = _N_DIAG // _NW
_LANES = 16

_mesh = plsc.VectorSubcoreMesh(core_axis_name="c", subcore_axis_name="s")


@functools.partial(
    pl.kernel,
    mesh=_mesh,
    out_type=jax.ShapeDtypeStruct((1, _F_PTS, _N_DIAG), jnp.float32),
    compiler_params=pltpu.CompilerParams(needs_layout_passes=False),
    scratch_types=[
        pltpu.VMEM((_F_PTS, 2, _NCHUNK), jnp.float32),
        pltpu.VMEM((_GRID_N, _GRID_N), jnp.float32),
        pltpu.VMEM((_F_PTS, _NCHUNK), jnp.float32),
        pltpu.VMEM((2, 2), jnp.float32),
    ],
)
def _sc_lookup(coords_hbm, grid_hbm, bounds_hbm, out_hbm,
               coords_v, grid_v, out_v, bounds_v):
    wid = lax.axis_index("s") * 2 + lax.axis_index("c")
    n0 = wid * _NCHUNK
    pltpu.sync_copy(coords_hbm.at[:, :, pl.ds(n0, _NCHUNK)], coords_v)
    pltpu.sync_copy(grid_hbm, grid_v)
    pltpu.sync_copy(bounds_hbm, bounds_v)

    zeros_i = jnp.zeros((_LANES,), jnp.int32)
    ones_i = zeros_i + 1
    m0 = plsc.load_gather(bounds_v, [zeros_i, zeros_i])
    big0 = plsc.load_gather(bounds_v, [zeros_i, ones_i])
    m1 = plsc.load_gather(bounds_v, [ones_i, zeros_i])
    big1 = plsc.load_gather(bounds_v, [ones_i, ones_i])
    gn = jnp.full((_LANES,), float(_GRID_N), jnp.float32)
    sx = gn / (big0 - m0)
    sy = gn / (big1 - m1)
    lim = jnp.full((_LANES,), float(_GRID_N - 1), jnp.float32)
    zero = jnp.zeros((_LANES,), jnp.float32)

    @plsc.parallel_loop(0, _F_PTS, unroll=2)
    def _loop(f):
        idx = []
        for g in range(_NCHUNK // _LANES):
            xs = coords_v[f, 0, pl.ds(g * _LANES, _LANES)]
            ys = coords_v[f, 1, pl.ds(g * _LANES, _LANES)]
            fx = jnp.minimum(jnp.maximum((xs - m0) * sx, zero), lim)
            fy = jnp.minimum(jnp.maximum((ys - m1) * sy, zero), lim)
            idx.append((fx.astype(jnp.int32), fy.astype(jnp.int32)))
        ws = [plsc.load_gather(grid_v, [ix, iy]) for ix, iy in idx]
        for g, w in enumerate(ws):
            out_v[f, pl.ds(g * _LANES, _LANES)] = w

    pltpu.sync_copy(out_v, out_hbm.at[0, :, pl.ds(n0, _NCHUNK)])


def kernel(diagrams, masks, grid, grid_bounds):
    del masks
    coords = jnp.transpose(diagrams, (1, 2, 0))
    out = _sc_lookup(coords, grid, grid_bounds)
    return jnp.transpose(out, (2, 1, 0))
